# baseline (speedup 1.0000x reference)
import functools

import jax
import jax.numpy as jnp
from jax import lax
from jax.experimental import pallas as pl
from jax.experimental.pallas import tpu as pltpu
from jax.experimental.pallas import tpu_sc as plsc

B = 16384
H = 64
C = 20
NC = 2
NS = 16
NW = NC * NS
BPW = B // NW
CB = 16
NCHUNK = BPW // CB
SUBW = 4
IDXW = CB * C // SUBW
TCW = 24

_mesh = plsc.VectorSubcoreMesh(
    core_axis_name="c", subcore_axis_name="s", num_cores=NC, num_subcores=NS
)


def _tc_gather_body(idx_smem, table_hbm, out_hbm, sem):
    def start(b):
        pltpu.make_async_copy(
            table_hbm.at[pl.ds(idx_smem[b], 1)],
            out_hbm.at[pl.ds(b, 1)], sem).start()

    def wait_one():
        pltpu.make_async_copy(
            table_hbm.at[pl.ds(0, 1)],
            out_hbm.at[pl.ds(0, 1)], sem).wait()

    def body(b, carry):
        start(b)

        @pl.when(b >= TCW)
        def _():
            wait_one()

        return carry

    lax.fori_loop(0, B, body, 0)

    def drain(b, carry):
        wait_one()
        return carry

    lax.fori_loop(0, TCW, drain, 0)


_tc_gather = pl.pallas_call(
    _tc_gather_body,
    grid_spec=pltpu.PrefetchScalarGridSpec(
        num_scalar_prefetch=1,
        grid=(1,),
        in_specs=[pl.BlockSpec(memory_space=pltpu.HBM)],
        out_specs=pl.BlockSpec(memory_space=pltpu.HBM),
        scratch_shapes=[pltpu.SemaphoreType.DMA],
    ),
    out_shape=jax.ShapeDtypeStruct((B, H), jnp.float32),
)


def _sc_body(cidx_hbm, t_hbm, oe_hbm, out_hbm,
             cidx_v, t_rows, c_buf, stage_v, out_v, sem_c0, sem_c1):
    wid = lax.axis_index("s") * NC + lax.axis_index("c")
    lane = lax.iota(jnp.int32, 16)

    pltpu.sync_copy(cidx_hbm.at[wid], cidx_v)
    pltpu.sync_copy(t_hbm.at[pl.ds(wid * BPW, BPW)], t_rows)

    sems = (sem_c0, sem_c1)
    for b2 in range(2):
        for i in range(SUBW):
            pltpu.async_copy(oe_hbm.at[cidx_v.at[b2, i]], c_buf.at[b2, i],
                             sems[b2])

    def compute_chunk(ci, b2):
        for i in range(SUBW):
            pltpu.make_async_copy(oe_hbm.at[pl.ds(0, IDXW)], c_buf.at[b2, i],
                                  sems[b2]).wait()

        def u_body(u, carry):
            sub = u // SUBW
            r0 = (u % SUBW) * C
            bb = ci * CB + u
            accs = [jnp.zeros((16,), jnp.float32) for _ in range(4)]
            for j in range(C):
                for k in range(4):
                    accs[k] = accs[k] + c_buf[b2, sub, r0 + j,
                                              pl.ds(k * 16, 16)]
            s = jnp.zeros((16,), jnp.float32)
            for k in range(4):
                s = s + accs[k] * t_rows[bb, pl.ds(k * 16, 16)]
            stage_v[u, :] = s
            return carry

        lax.fori_loop(0, CB, u_body, 0)
        total = jnp.zeros((16,), jnp.float32)
        for col in range(16):
            total = total + plsc.load_gather(
                stage_v, [lane, jnp.full((16,), col, jnp.int32)])
        out_v[pl.ds(ci * CB, CB)] = total * (1.0 / C)

        @pl.when(ci + 2 < NCHUNK)
        def _():
            for i in range(SUBW):
                pltpu.async_copy(oe_hbm.at[cidx_v.at[ci + 2, i]],
                                 c_buf.at[b2, i], sems[b2])

    def outer(g, carry):
        for b2 in range(2):
            compute_chunk(2 * g + b2, b2)
        return carry

    lax.fori_loop(0, NCHUNK // 2, outer, 0)
    pltpu.sync_copy(out_v, out_hbm.at[pl.ds(wid * BPW, BPW)])


_skipgram_sc = pl.kernel(
    _sc_body,
    out_type=jax.ShapeDtypeStruct((B,), jnp.float32),
    mesh=_mesh,
    compiler_params=pltpu.CompilerParams(
        needs_layout_passes=False, use_tc_tiling_on_sc=False),
    scratch_types=[
        pltpu.VMEM((NCHUNK, SUBW, IDXW), jnp.int32),
        pltpu.VMEM((BPW, H), jnp.float32),
        pltpu.VMEM((2, SUBW, IDXW, H), jnp.float32),
        pltpu.VMEM((16, 16), jnp.float32),
        pltpu.VMEM((BPW,), jnp.float32),
        pltpu.SemaphoreType.DMA,
        pltpu.SemaphoreType.DMA,
    ],
)


def kernel(x, in_embedding, out_embedding):
    xi = x.astype(jnp.int32)
    tgt_idx = xi[:, 0]
    ctx_idx = xi[:, 1:].reshape(NW, NCHUNK, SUBW, IDXW)
    t_rows = _tc_gather(tgt_idx, in_embedding)
    out = _skipgram_sc(ctx_idx, t_rows, out_embedding)
    return out.reshape(B, 1)


# speedup vs baseline: 1.7704x; 1.7704x over previous
import jax
import jax.numpy as jnp
from jax import lax
from jax.experimental import pallas as pl
from jax.experimental.pallas import tpu as pltpu
from jax.experimental.pallas import tpu_sc as plsc

B = 16384
H = 64
C = 20
NC = 2
NS = 16
NW = NC * NS
BPW = B // NW
CB = 16
NCHUNK = BPW // CB
SUBW = 4
IDXW = CB * C // SUBW

_mesh = plsc.VectorSubcoreMesh(
    core_axis_name="c", subcore_axis_name="s", num_cores=NC, num_subcores=NS
)


def _sc_body(cidx_hbm, t_hbm, oe_hbm, out_hbm,
             cidx_v, t_rows, c_buf, stage_v, out_v, sem_c0, sem_c1):
    wid = lax.axis_index("s") * NC + lax.axis_index("c")
    lane = lax.iota(jnp.int32, 16)

    pltpu.sync_copy(cidx_hbm.at[wid], cidx_v)
    pltpu.sync_copy(t_hbm.at[pl.ds(wid * BPW, BPW)], t_rows)

    sems = (sem_c0, sem_c1)
    for b2 in range(2):
        for i in range(SUBW):
            pltpu.async_copy(oe_hbm.at[cidx_v.at[b2, i]], c_buf.at[b2, i],
                             sems[b2])

    def compute_chunk(ci, b2):
        for i in range(SUBW):
            pltpu.make_async_copy(oe_hbm.at[pl.ds(0, IDXW)], c_buf.at[b2, i],
                                  sems[b2]).wait()

        def u_body(u, carry):
            sub = u // SUBW
            r0 = (u % SUBW) * C
            bb = ci * CB + u
            accs = [jnp.zeros((16,), jnp.float32) for _ in range(4)]
            for j in range(C):
                for k in range(4):
                    accs[k] = accs[k] + c_buf[b2, sub, r0 + j,
                                              pl.ds(k * 16, 16)]
            s = jnp.zeros((16,), jnp.float32)
            for k in range(4):
                s = s + accs[k] * t_rows[bb, pl.ds(k * 16, 16)]
            stage_v[u, :] = s
            return carry

        lax.fori_loop(0, CB, u_body, 0)
        total = jnp.zeros((16,), jnp.float32)
        for col in range(16):
            total = total + plsc.load_gather(
                stage_v, [lane, jnp.full((16,), col, jnp.int32)])
        out_v[pl.ds(ci * CB, CB)] = total * (1.0 / C)

        @pl.when(ci + 2 < NCHUNK)
        def _():
            for i in range(SUBW):
                pltpu.async_copy(oe_hbm.at[cidx_v.at[ci + 2, i]],
                                 c_buf.at[b2, i], sems[b2])

    def outer(g, carry):
        for b2 in range(2):
            compute_chunk(2 * g + b2, b2)
        return carry

    lax.fori_loop(0, NCHUNK // 2, outer, 0)
    pltpu.sync_copy(out_v, out_hbm.at[pl.ds(wid * BPW, BPW)])


_skipgram_sc = pl.kernel(
    _sc_body,
    out_type=jax.ShapeDtypeStruct((B,), jnp.float32),
    mesh=_mesh,
    compiler_params=pltpu.CompilerParams(
        needs_layout_passes=False, use_tc_tiling_on_sc=False),
    scratch_types=[
        pltpu.VMEM((NCHUNK, SUBW, IDXW), jnp.int32),
        pltpu.VMEM((BPW, H), jnp.float32),
        pltpu.VMEM((2, SUBW, IDXW, H), jnp.float32),
        pltpu.VMEM((16, 16), jnp.float32),
        pltpu.VMEM((BPW,), jnp.float32),
        pltpu.SemaphoreType.DMA,
        pltpu.SemaphoreType.DMA,
    ],
)


def kernel(x, in_embedding, out_embedding):
    xi = x.astype(jnp.int32)
    ctx_idx = xi[:, 1:].reshape(NW, NCHUNK, SUBW, IDXW)
    t_rows = jnp.take(in_embedding, xi[:, 0], axis=0)
    out = _skipgram_sc(ctx_idx, t_rows, out_embedding)
    return out.reshape(B, 1)
